# baseline (speedup 1.0000x reference)
import functools

import jax
import jax.numpy as jnp
from jax import lax
from jax.experimental import pallas as pl
from jax.experimental.pallas import tpu as pltpu
from jax.experimental.pallas import tpu_sc as plsc

N = 10000
E = 320000
D = 128
D_OUT = 64

NC = 2
NS = 16
CHUNK = 128
SPLIT = 4
G = 40
CHUNKS = -(-E // (NS * CHUNK * G)) * G
GROUPS = CHUNKS // G
EPT = CHUNKS * CHUNK
PAD = NS * EPT - E
ROWS_PER_TILE = 632
NPAD = N + 16

_sc_mesh = plsc.VectorSubcoreMesh(core_axis_name="c", subcore_axis_name="s")


@functools.partial(
    pl.kernel,
    out_type=jax.ShapeDtypeStruct((NC, N, D), jnp.float32),
    mesh=_sc_mesh,
    scratch_types=[
        pltpu.VMEM((G, CHUNK), jnp.int32),
        pltpu.VMEM((G, CHUNK), jnp.int32),
        pltpu.VMEM((CHUNK, D), jnp.float32),
        pltpu.VMEM((CHUNK, D), jnp.float32),
        pltpu.VMEM_SHARED((NPAD, D), jnp.float32),
        pltpu.SemaphoreType.DMA,
        pltpu.SemaphoreType.DMA,
    ],
)
def _spmm(x_hbm, src_hbm, dst_hbm, out_hbm, src_v, dst_v, rows0, rows1,
          acc_sh, sem0, sem1):
    c = lax.axis_index("c")
    s = lax.axis_index("s")
    row0 = pl.multiple_of(
        jnp.where(s == NS - 1, N - ROWS_PER_TILE, s * ROWS_PER_TILE), 8)
    pltpu.sync_copy(
        x_hbm.at[pl.ds(row0, ROWS_PER_TILE)],
        acc_sh.at[pl.ds(row0, ROWS_PER_TILE)],
    )
    plsc.subcore_barrier()

    sub = CHUNK // SPLIT

    def start_gather(t, buf, sem):
        for i in range(SPLIT):
            pltpu.async_copy(
                x_hbm.at[src_v.at[t, pl.ds(i * sub, sub)]],
                buf.at[pl.ds(i * sub, sub)], sem)

    def wait_gather(t, buf, sem):
        pltpu.make_async_copy(x_hbm.at[src_v.at[t]], buf, sem).wait()

    def group(g, carry):
        g0 = pl.multiple_of(g * G, 8)
        pltpu.sync_copy(src_hbm.at[c, s, pl.ds(g0, G)], src_v)
        pltpu.sync_copy(dst_hbm.at[c, s, pl.ds(g0, G)], dst_v)
        start_gather(0, rows0, sem0)

        def pair(p, carry2):
            t = 2 * p
            start_gather(t + 1, rows1, sem1)
            wait_gather(t, rows0, sem0)
            pltpu.sync_copy(rows0, acc_sh.at[dst_v.at[t]], add=True)

            @pl.when(t + 2 < G)
            def _():
                start_gather(t + 2, rows0, sem0)

            wait_gather(t + 1, rows1, sem1)
            pltpu.sync_copy(rows1, acc_sh.at[dst_v.at[t + 1]], add=True)
            return carry2

        lax.fori_loop(0, G // 2, pair, 0)
        return carry

    lax.fori_loop(0, GROUPS, group, 0)
    plsc.subcore_barrier()
    pltpu.sync_copy(
        acc_sh.at[pl.ds(row0, ROWS_PER_TILE)],
        out_hbm.at[c, pl.ds(row0, ROWS_PER_TILE)],
    )


BLK = 2000


def _combine_body(eps_ref, x_ref, h0_ref, h1_ref, w0_ref, w1_ref, o_ref):
    o_ref[...] = (
        (1.0 + eps_ref[0, 0]) * x_ref[...]
        + jnp.dot(h0_ref[...], w0_ref[...], preferred_element_type=jnp.float32)
        + jnp.dot(h1_ref[...], w1_ref[...], preferred_element_type=jnp.float32)
    )


def _combine_head_body(eps_ref, x_ref, h0_ref, h1_ref, w0_ref, w1_ref,
                       wh_ref, o_ref):
    t = (
        (1.0 + eps_ref[0, 0]) * x_ref[...]
        + jnp.dot(h0_ref[...], w0_ref[...], preferred_element_type=jnp.float32)
        + jnp.dot(h1_ref[...], w1_ref[...], preferred_element_type=jnp.float32)
    )
    o_ref[...] = jnp.dot(t, wh_ref[...], preferred_element_type=jnp.float32)


def _row_spec(i):
    return (i, 0)


def _rep_spec(i):
    return (0, 0)


_combine = pl.pallas_call(
    _combine_body,
    grid=(N // BLK,),
    in_specs=[
        pl.BlockSpec(memory_space=pltpu.SMEM),
        pl.BlockSpec((BLK, D), _row_spec),
        pl.BlockSpec((BLK, D), _row_spec),
        pl.BlockSpec((BLK, D), _row_spec),
        pl.BlockSpec((D, D), _rep_spec),
        pl.BlockSpec((D, D), _rep_spec),
    ],
    out_specs=pl.BlockSpec((BLK, D), _row_spec),
    out_shape=jax.ShapeDtypeStruct((N, D), jnp.float32),
)

_combine_head = pl.pallas_call(
    _combine_head_body,
    grid=(N // BLK,),
    in_specs=[
        pl.BlockSpec(memory_space=pltpu.SMEM),
        pl.BlockSpec((BLK, D), _row_spec),
        pl.BlockSpec((BLK, D), _row_spec),
        pl.BlockSpec((BLK, D), _row_spec),
        pl.BlockSpec((D, D), _rep_spec),
        pl.BlockSpec((D, D), _rep_spec),
        pl.BlockSpec((D, D_OUT), _rep_spec),
    ],
    out_specs=pl.BlockSpec((BLK, D_OUT), _row_spec),
    out_shape=jax.ShapeDtypeStruct((N, D_OUT), jnp.float32),
)


def _prep_idx(src, dst):
    src = jnp.concatenate(
        [src.astype(jnp.int32), jnp.zeros((PAD,), jnp.int32)])
    dst = jnp.concatenate(
        [dst.astype(jnp.int32), jnp.full((PAD,), N, jnp.int32)])
    return src.reshape(NS, CHUNKS, CHUNK), dst.reshape(NS, CHUNKS, CHUNK)


@jax.jit
def kernel(x, agg_scatter_index_0, agg_node_index_0, agg_scatter_index_1,
           agg_node_index_1, eps0, eps1, W_l0_h0, W_l0_h1, W_l1_h0, W_l1_h1,
           W_head):
    s0, d0 = _prep_idx(agg_scatter_index_0, agg_node_index_0)
    s1, d1 = _prep_idx(agg_scatter_index_1, agg_node_index_1)
    src_all = jnp.stack([s0, s1])
    dst_all = jnp.stack([d0, d1])
    e0 = eps0.reshape(1, 1)
    e1 = eps1.reshape(1, 1)

    h = _spmm(x, src_all, dst_all)
    x1 = _combine(e0, x, h[0], h[1], W_l0_h0, W_l0_h1)
    h = _spmm(x1, src_all, dst_all)
    return _combine_head(e1, x1, h[0], h[1], W_l1_h0, W_l1_h1, W_head)


# speedup vs baseline: 1.0463x; 1.0463x over previous
import functools

import jax
import jax.numpy as jnp
from jax import lax
from jax.experimental import pallas as pl
from jax.experimental.pallas import tpu as pltpu
from jax.experimental.pallas import tpu_sc as plsc

N = 10000
E = 320000
D = 128
D_OUT = 64

NC = 2
NS = 16
CH = 32
G = 8
CHUNKS = -(-E // (NS * CH * G)) * G
GROUPS = CHUNKS // G
EPT = CHUNKS * CH
PAD = NS * EPT - E
HALFN = N // 2
NPAD = HALFN + 16
RPT = 632
RPT2 = 320

_sc_mesh = plsc.VectorSubcoreMesh(core_axis_name="c", subcore_axis_name="s")


@functools.partial(
    pl.kernel,
    out_type=jax.ShapeDtypeStruct((2, N, D), jnp.float32),
    mesh=_sc_mesh,
    scratch_types=[
        pltpu.VMEM((G, CH), jnp.int32),
        pltpu.VMEM((G, CH), jnp.int32),
        pltpu.VMEM((CH, D), jnp.float32),
        pltpu.VMEM((CH, D), jnp.float32),
        pltpu.VMEM_SHARED((N, D), jnp.float32),
        pltpu.VMEM_SHARED((NPAD, D), jnp.float32),
        pltpu.SemaphoreType.DMA,
        pltpu.SemaphoreType.DMA,
    ],
)
def _spmm(x_hbm, src_hbm, dst_hbm, out_hbm, src_v, dst_v, rows0, rows1,
          x_sh, acc, sem0, sem1):
    c = lax.axis_index("c")
    s = lax.axis_index("s")
    base = pl.multiple_of(c * HALFN, 8)
    sink = HALFN + s
    row0 = pl.multiple_of(jnp.where(s == NS - 1, N - RPT, s * RPT), 8)
    hrow0 = pl.multiple_of(
        jnp.where(s == NS - 1, HALFN - RPT2, s * RPT2), 8)

    pltpu.sync_copy(x_hbm.at[pl.ds(row0, RPT)], x_sh.at[pl.ds(row0, RPT)])

    def init_acc():
        pltpu.sync_copy(
            x_hbm.at[pl.ds(pl.multiple_of(base + hrow0, 8), RPT2)],
            acc.at[pl.ds(hrow0, RPT2)])

    init_acc()
    plsc.subcore_barrier()

    def run_hop(hop):
        def group(g, carry):
            g0 = pl.multiple_of(g * G, 8)
            pltpu.sync_copy(src_hbm.at[hop, s, pl.ds(g0, G)], src_v)
            pltpu.sync_copy(dst_hbm.at[hop, s, pl.ds(g0, G)], dst_v)

            def clip_t(t, cc):
                def clip_j(j, cc2):
                    v = dst_v[t, pl.ds(j * 16, 16)]
                    lv = v - base
                    ok = (lv >= 0) & (lv < HALFN)
                    dst_v[t, pl.ds(j * 16, 16)] = jnp.where(ok, lv, sink)
                    return cc2

                return lax.fori_loop(0, CH // 16, clip_j, cc)

            lax.fori_loop(0, G, clip_t, carry)
            pltpu.async_copy(x_sh.at[src_v.at[0]], rows0, sem0)

            def pair(p, carry2):
                t = 2 * p
                pltpu.async_copy(x_sh.at[src_v.at[t + 1]], rows1, sem1)
                pltpu.make_async_copy(x_sh.at[src_v.at[t]], rows0,
                                      sem0).wait()
                pltpu.sync_copy(rows0, acc.at[dst_v.at[t]], add=True)

                @pl.when(t + 2 < G)
                def _():
                    pltpu.async_copy(x_sh.at[src_v.at[t + 2]], rows0, sem0)

                pltpu.make_async_copy(x_sh.at[src_v.at[t + 1]], rows1,
                                      sem1).wait()
                pltpu.sync_copy(rows1, acc.at[dst_v.at[t + 1]], add=True)
                return carry2

            lax.fori_loop(0, G // 2, pair, 0)
            return carry

        lax.fori_loop(0, GROUPS, group, 0)

    run_hop(0)
    plsc.subcore_barrier()
    pltpu.sync_copy(
        acc.at[pl.ds(hrow0, RPT2)],
        out_hbm.at[0, pl.ds(pl.multiple_of(base + hrow0, 8), RPT2)])
    plsc.subcore_barrier()
    init_acc()
    plsc.subcore_barrier()
    run_hop(1)
    plsc.subcore_barrier()
    pltpu.sync_copy(
        acc.at[pl.ds(hrow0, RPT2)],
        out_hbm.at[1, pl.ds(pl.multiple_of(base + hrow0, 8), RPT2)])


BLK = 2000


def _combine_body(eps_ref, x_ref, h_ref, w0_ref, w1_ref, o_ref):
    o_ref[...] = (
        (1.0 + eps_ref[0, 0]) * x_ref[...]
        + jnp.dot(h_ref[0], w0_ref[...], preferred_element_type=jnp.float32)
        + jnp.dot(h_ref[1], w1_ref[...], preferred_element_type=jnp.float32)
    )


def _combine_head_body(eps_ref, x_ref, h_ref, w0_ref, w1_ref, wh_ref, o_ref):
    t = (
        (1.0 + eps_ref[0, 0]) * x_ref[...]
        + jnp.dot(h_ref[0], w0_ref[...], preferred_element_type=jnp.float32)
        + jnp.dot(h_ref[1], w1_ref[...], preferred_element_type=jnp.float32)
    )
    o_ref[...] = jnp.dot(t, wh_ref[...], preferred_element_type=jnp.float32)


def _row_spec(i):
    return (i, 0)


def _rep_spec(i):
    return (0, 0)


def _h_spec(i):
    return (0, i, 0)


_combine = pl.pallas_call(
    _combine_body,
    grid=(N // BLK,),
    in_specs=[
        pl.BlockSpec(memory_space=pltpu.SMEM),
        pl.BlockSpec((BLK, D), _row_spec),
        pl.BlockSpec((2, BLK, D), _h_spec),
        pl.BlockSpec((D, D), _rep_spec),
        pl.BlockSpec((D, D), _rep_spec),
    ],
    out_specs=pl.BlockSpec((BLK, D), _row_spec),
    out_shape=jax.ShapeDtypeStruct((N, D), jnp.float32),
)

_combine_head = pl.pallas_call(
    _combine_head_body,
    grid=(N // BLK,),
    in_specs=[
        pl.BlockSpec(memory_space=pltpu.SMEM),
        pl.BlockSpec((BLK, D), _row_spec),
        pl.BlockSpec((2, BLK, D), _h_spec),
        pl.BlockSpec((D, D), _rep_spec),
        pl.BlockSpec((D, D), _rep_spec),
        pl.BlockSpec((D, D_OUT), _rep_spec),
    ],
    out_specs=pl.BlockSpec((BLK, D_OUT), _row_spec),
    out_shape=jax.ShapeDtypeStruct((N, D_OUT), jnp.float32),
)


def _prep_idx(src, dst):
    src = jnp.concatenate(
        [src.astype(jnp.int32), jnp.zeros((PAD,), jnp.int32)])
    dst = jnp.concatenate(
        [dst.astype(jnp.int32), jnp.full((PAD,), N, jnp.int32)])
    return src.reshape(NS, CHUNKS, CH), dst.reshape(NS, CHUNKS, CH)


@jax.jit
def kernel(x, agg_scatter_index_0, agg_node_index_0, agg_scatter_index_1,
           agg_node_index_1, eps0, eps1, W_l0_h0, W_l0_h1, W_l1_h0, W_l1_h1,
           W_head):
    s0, d0 = _prep_idx(agg_scatter_index_0, agg_node_index_0)
    s1, d1 = _prep_idx(agg_scatter_index_1, agg_node_index_1)
    src_all = jnp.stack([s0, s1])
    dst_all = jnp.stack([d0, d1])
    e0 = eps0.reshape(1, 1)
    e1 = eps1.reshape(1, 1)

    h = _spmm(x, src_all, dst_all)
    x1 = _combine(e0, x, h, W_l0_h0, W_l0_h1)
    h = _spmm(x1, src_all, dst_all)
    return _combine_head(e1, x1, h, W_l1_h0, W_l1_h1, W_head)
